# Initial kernel scaffold; baseline (speedup 1.0000x reference)
import functools

import jax
import jax.numpy as jnp
from jax import lax
from jax.experimental import pallas as pl
from jax.experimental.pallas import tpu as pltpu
from jax.experimental.pallas import tpu_sc as plsc

NUM_CORES = 2
NUM_SUBCORES = 16
NW = NUM_CORES * NUM_SUBCORES
CHUNK = 128
D = 64


@functools.lru_cache(maxsize=None)
def _build(B):
    n_per_w = B // NW
    n_chunks = n_per_w // CHUNK
    n_pairs = n_chunks // 2
    mesh = plsc.VectorSubcoreMesh(core_axis_name="c", subcore_axis_name="s")

    @functools.partial(
        pl.kernel,
        mesh=mesh,
        out_type=jax.ShapeDtypeStruct((B, D), jnp.float32),
        scratch_types=[
            pltpu.VMEM((n_chunks, CHUNK), jnp.int32),
            pltpu.VMEM((CHUNK, D), jnp.float32),
            pltpu.VMEM((CHUNK, D), jnp.float32),
            pltpu.SemaphoreType.DMA,
            pltpu.SemaphoreType.DMA,
        ],
    )
    def emb(table_hbm, idx_hbm, out_hbm, idx_v, rows0, rows1, sem0, sem1):
        wid = lax.axis_index("s") * NUM_CORES + lax.axis_index("c")
        base = wid * n_per_w
        pltpu.sync_copy(idx_hbm.at[wid], idx_v)
        pltpu.make_async_copy(table_hbm.at[idx_v.at[0]], rows0, sem0).start()

        def pair_body(p, _):
            c0 = 2 * p
            pltpu.make_async_copy(
                table_hbm.at[idx_v.at[c0 + 1]], rows1, sem1
            ).start()
            pltpu.make_async_copy(
                table_hbm.at[idx_v.at[0]], rows0, sem0
            ).wait()
            pltpu.sync_copy(
                rows0, out_hbm.at[pl.ds(base + c0 * CHUNK, CHUNK)]
            )

            @pl.when(c0 + 2 < n_chunks)
            def _():
                pltpu.make_async_copy(
                    table_hbm.at[idx_v.at[c0 + 2]], rows0, sem0
                ).start()

            pltpu.make_async_copy(
                table_hbm.at[idx_v.at[0]], rows1, sem1
            ).wait()
            pltpu.sync_copy(
                rows1, out_hbm.at[pl.ds(base + (c0 + 1) * CHUNK, CHUNK)]
            )
            return 0

        lax.fori_loop(0, n_pairs, pair_body, 0)

    return emb


def kernel(token_ids, weights):
    orig_shape = token_ids.shape
    idx = token_ids.reshape(-1).astype(jnp.int32)
    B = idx.shape[0]
    idx3 = idx.reshape(NW, (B // NW) // CHUNK, CHUNK)
    out = _build(B)(weights, idx3)
    return out.reshape(*orig_shape, D)


# speedup vs baseline: 1.8403x; 1.8403x over previous
import functools

import jax
import jax.numpy as jnp
from jax import lax
from jax.experimental import pallas as pl
from jax.experimental.pallas import tpu as pltpu
from jax.experimental.pallas import tpu_sc as plsc

NUM_CORES = 2
NUM_SUBCORES = 16
NW = NUM_CORES * NUM_SUBCORES
CHUNK = 128
D = 64


@functools.lru_cache(maxsize=None)
def _build(B):
    n_per_w = B // NW
    n_chunks = n_per_w // CHUNK
    n_pairs = n_chunks // 2
    mesh = plsc.VectorSubcoreMesh(core_axis_name="c", subcore_axis_name="s")

    @functools.partial(
        pl.kernel,
        mesh=mesh,
        out_type=jax.ShapeDtypeStruct((B, D), jnp.float32),
        compiler_params=pltpu.CompilerParams(use_tc_tiling_on_sc=False),
        scratch_types=[
            pltpu.VMEM((n_chunks, CHUNK), jnp.int32),
            pltpu.VMEM((CHUNK, D), jnp.float32),
            pltpu.VMEM((CHUNK, D), jnp.float32),
            pltpu.SemaphoreType.DMA,
            pltpu.SemaphoreType.DMA,
        ],
    )
    def emb(table_hbm, idx_hbm, out_hbm, idx_v, rows0, rows1, sem0, sem1):
        wid = lax.axis_index("s") * NUM_CORES + lax.axis_index("c")
        base = wid * n_per_w
        pltpu.sync_copy(idx_hbm.at[wid], idx_v)
        pltpu.make_async_copy(table_hbm.at[idx_v.at[0]], rows0, sem0).start()

        def pair_body(p, _):
            c0 = 2 * p
            pltpu.make_async_copy(
                table_hbm.at[idx_v.at[c0 + 1]], rows1, sem1
            ).start()
            pltpu.make_async_copy(
                table_hbm.at[idx_v.at[0]], rows0, sem0
            ).wait()
            pltpu.sync_copy(
                rows0, out_hbm.at[pl.ds(base + c0 * CHUNK, CHUNK)]
            )

            @pl.when(c0 + 2 < n_chunks)
            def _():
                pltpu.make_async_copy(
                    table_hbm.at[idx_v.at[c0 + 2]], rows0, sem0
                ).start()

            pltpu.make_async_copy(
                table_hbm.at[idx_v.at[0]], rows1, sem1
            ).wait()
            pltpu.sync_copy(
                rows1, out_hbm.at[pl.ds(base + (c0 + 1) * CHUNK, CHUNK)]
            )
            return 0

        lax.fori_loop(0, n_pairs, pair_body, 0)

    return emb


def kernel(token_ids, weights):
    orig_shape = token_ids.shape
    idx = token_ids.reshape(-1).astype(jnp.int32)
    B = idx.shape[0]
    idx3 = idx.reshape(NW, (B // NW) // CHUNK, CHUNK)
    out = _build(B)(weights, idx3)
    return out.reshape(*orig_shape, D)


# speedup vs baseline: 1.8762x; 1.0195x over previous
import functools

import jax
import jax.numpy as jnp
from jax import lax
from jax.experimental import pallas as pl
from jax.experimental.pallas import tpu as pltpu
from jax.experimental.pallas import tpu_sc as plsc

NUM_CORES = 2
NUM_SUBCORES = 16
NW = NUM_CORES * NUM_SUBCORES
CHUNK = 128
D = 64


NBUF = 4


@functools.lru_cache(maxsize=None)
def _build(B):
    n_per_w = B // NW
    n_chunks = n_per_w // CHUNK
    assert n_chunks % NBUF == 0
    mesh = plsc.VectorSubcoreMesh(core_axis_name="c", subcore_axis_name="s")

    @functools.partial(
        pl.kernel,
        mesh=mesh,
        out_type=jax.ShapeDtypeStruct((B, D), jnp.float32),
        compiler_params=pltpu.CompilerParams(use_tc_tiling_on_sc=False),
        scratch_types=[
            pltpu.VMEM((n_chunks, CHUNK), jnp.int32),
        ]
        + [pltpu.VMEM((CHUNK, D), jnp.float32) for _ in range(NBUF)]
        + [pltpu.SemaphoreType.DMA for _ in range(2 * NBUF)],
    )
    def emb(table_hbm, idx_hbm, out_hbm, idx_v, *bufs_sems):
        rows = bufs_sems[:NBUF]
        gsem = bufs_sems[NBUF : 2 * NBUF]
        ssem = bufs_sems[2 * NBUF :]
        wid = lax.axis_index("s") * NUM_CORES + lax.axis_index("c")
        base = wid * n_per_w
        pltpu.sync_copy(idx_hbm.at[wid], idx_v)
        for b in range(NBUF):
            pltpu.make_async_copy(
                table_hbm.at[idx_v.at[b]], rows[b], gsem[b]
            ).start()

        def ring_body(j, _):
            c = j * NBUF
            for b in range(NBUF):
                i = c + b
                pltpu.make_async_copy(
                    table_hbm.at[idx_v.at[0]], rows[b], gsem[b]
                ).wait()
                pltpu.make_async_copy(
                    rows[b], out_hbm.at[pl.ds(base + i * CHUNK, CHUNK)], ssem[b]
                ).start()

                @pl.when(i + NBUF < n_chunks)
                def _(b=b, i=i):
                    pltpu.make_async_copy(
                        rows[b],
                        out_hbm.at[pl.ds(base, CHUNK)],
                        ssem[b],
                    ).wait()
                    pltpu.make_async_copy(
                        table_hbm.at[idx_v.at[i + NBUF]], rows[b], gsem[b]
                    ).start()

            return 0

        lax.fori_loop(0, n_chunks // NBUF, ring_body, 0)
        for b in range(NBUF):
            pltpu.make_async_copy(
                rows[b], out_hbm.at[pl.ds(base, CHUNK)], ssem[b]
            ).wait()

    return emb


def kernel(token_ids, weights):
    orig_shape = token_ids.shape
    idx = token_ids.reshape(-1).astype(jnp.int32)
    B = idx.shape[0]
    idx3 = idx.reshape(NW, (B // NW) // CHUNK, CHUNK)
    out = _build(B)(weights, idx3)
    return out.reshape(*orig_shape, D)
